# Initial kernel scaffold; baseline (speedup 1.0000x reference)
import jax
import jax.numpy as jnp
from jax.experimental import pallas as pl


def kernel(x, W1f, W1d, W2f, W2d, Wlf, Wld):
    raise NotImplementedError("write your pallas kernel here")


# speedup vs baseline: 8.1951x; 8.1951x over previous
import functools
import jax
import jax.numpy as jnp
from jax import lax
from jax.experimental import pallas as pl
from jax.experimental.pallas import tpu as pltpu
from jax.experimental.pallas import tpu_sc as plsc

K = 20
NEG = 0.8
EPS = 1e-6
NC, NS, L = 2, 16, 16
NW = NC * NS


def _topk_body(xff_ref, xfb_ref, idx_ref):
    xff = xff_ref[0]
    xfb = xfb_ref[0]
    tn = xfb.shape[1]
    n = xff.shape[1]
    inner = jax.lax.dot_general(
        xfb, xff, (((0,), (0,)), ((), ())),
        preferred_element_type=jnp.float32,
        precision=jax.lax.Precision.DEFAULT)
    xx = jnp.sum(xff * xff, axis=0, keepdims=True)
    xxr = jnp.sum(xfb * xfb, axis=0)[:, None]
    work = (-xx - (-2.0 * inner)) - xxr
    iota = lax.broadcasted_iota(jnp.int32, (tn, n), 1)
    cols = []
    for _ in range(K):
        m = jnp.max(work, axis=1, keepdims=True)
        cand = jnp.min(jnp.where(work >= m, iota, n), axis=1, keepdims=True)
        cols.append(cand)
        work = jnp.where(iota == cand, -jnp.inf, work)
    idx_ref[0] = jnp.concatenate(cols, axis=1)


def _topk(xfp, tn=256):
    b, cp, n = xfp.shape
    grid = (b, n // tn)
    return pl.pallas_call(
        _topk_body,
        grid=grid,
        in_specs=[
            pl.BlockSpec((1, cp, n), lambda i, j: (i, 0, 0)),
            pl.BlockSpec((1, cp, tn), lambda i, j: (i, 0, j)),
        ],
        out_specs=pl.BlockSpec((1, tn, K), lambda i, j: (i, j, 0)),
        out_shape=jax.ShapeDtypeStruct((b, n, K), jnp.int32),
    )(xfp, xfp)


def _pre_body(m_ref, x_ref, g_ref):
    for v in range(3):
        xv = x_ref[0, :, v, :]
        g_ref[0, :, v, :] = jax.lax.dot_general(
            m_ref[...], xv, (((1,), (0,)), ((), ())),
            preferred_element_type=jnp.float32,
            precision=jax.lax.Precision.DEFAULT)


def _precompute(xs, m):
    b, c, _, n = xs.shape
    return pl.pallas_call(
        _pre_body,
        grid=(b,),
        in_specs=[
            pl.BlockSpec((84, c), lambda i: (0, 0)),
            pl.BlockSpec((1, c, 3, n), lambda i: (i, 0, 0, 0)),
        ],
        out_specs=pl.BlockSpec((1, 84, 3, n), lambda i: (i, 0, 0, 0)),
        out_shape=jax.ShapeDtypeStruct((b, 84, 3, n), jnp.float32),
    )(m, xs)


def _gather_call(idx_flat, xr, nsplit):
    b, r, n = xr.shape
    nseg = n // nsplit
    ntask = b * nsplit
    mesh = plsc.VectorSubcoreMesh(core_axis_name="c", subcore_axis_name="s")

    @functools.partial(
        pl.kernel,
        out_type=jax.ShapeDtypeStruct((b, r, K, n), jnp.float32),
        mesh=mesh,
        scratch_types=[
            pltpu.VMEM((nseg * K,), jnp.int32),
            pltpu.VMEM((r, n), jnp.float32),
            pltpu.VMEM((r, nseg), jnp.float32),
        ],
        compiler_params=pltpu.CompilerParams(needs_layout_passes=False),
    )
    def gat(idx_hbm, xr_hbm, fd_hbm, idx_v, table_v, stage_v):
        w = lax.axis_index("s") * NC + lax.axis_index("c")
        t0 = (w * ntask) // NW
        t1 = ((w + 1) * ntask) // NW
        iota = lax.iota(jnp.int32, L)

        def task(t, carry):
            bb = t // nsplit
            n0 = (t % nsplit) * nseg
            pltpu.sync_copy(idx_hbm.at[bb, pl.ds(n0 * K, nseg * K)], idx_v)
            pltpu.sync_copy(xr_hbm.at[bb], table_v)

            def per_k(k, carry2):
                def grp(gg, carry3):
                    jv = plsc.load_gather(idx_v, [(gg * L + iota) * K + k])
                    lsl = pl.ds(gg * L, L)
                    gsl = pl.ds(n0 + gg * L, L)
                    for rr in range(r):
                        gv = plsc.load_gather(
                            table_v, [jnp.full((L,), rr, jnp.int32), jv])
                        stage_v[rr, lsl] = gv - table_v[rr, gsl]
                    return carry3

                lax.fori_loop(0, nseg // L, grp, 0)
                pltpu.sync_copy(stage_v, fd_hbm.at[bb, :, k, pl.ds(n0, nseg)])
                return carry2

            lax.fori_loop(0, K, per_k, 0)
            return carry

        lax.fori_loop(t0, t1, task, 0)

    return gat(idx_flat, xr)


def _edge1_body(wblk_ref, wbblk_ref, x_ref, fd_ref, h_ref):
    tn = x_ref.shape[3]
    r3 = x_ref.shape[1] * 3
    xall = x_ref[0].reshape(r3, tn)
    pbdb = jax.lax.dot_general(
        wbblk_ref[...], xall, (((1,), (0,)), ((), ())),
        preferred_element_type=jnp.float32,
        precision=jax.lax.Precision.DEFAULT)
    acc = [jnp.zeros((21, tn), jnp.float32) for _ in range(3)]
    for k in range(K):
        fd = fd_ref[0, :, k, :]
        pada = jax.lax.dot_general(
            wblk_ref[...], fd, (((1,), (0,)), ((), ())),
            preferred_element_type=jnp.float32,
            precision=jax.lax.Precision.DEFAULT) + pbdb
        ps = [pada[v * 42:v * 42 + 21, :] for v in range(3)]
        ds = [pada[v * 42 + 21:v * 42 + 42, :] for v in range(3)]
        dot = (ps[0] * ds[0] + ps[1] * ds[1]) + ps[2] * ds[2]
        dsq = (ds[0] * ds[0] + ds[1] * ds[1]) + ds[2] * ds[2]
        dotq = dot / (dsq + EPS)
        msk = (dot >= 0.0).astype(jnp.float32)
        for v in range(3):
            inner = msk * ps[v] + (1.0 - msk) * (ps[v] - dotq * ds[v])
            acc[v] = acc[v] + (0.2 * ps[v] + 0.8 * inner)
    for v in range(3):
        h_ref[0, :, v, :] = acc[v] / jnp.float32(K)


def _edge1(x, fd, wblk, wbblk):
    b, _, _, n = x.shape
    tn = 256
    return pl.pallas_call(
        _edge1_body,
        grid=(b, n // tn),
        in_specs=[
            pl.BlockSpec((126, 12), lambda i, j: (0, 0)),
            pl.BlockSpec((126, 12), lambda i, j: (0, 0)),
            pl.BlockSpec((1, 4, 3, tn), lambda i, j: (i, 0, 0, j)),
            pl.BlockSpec((1, 12, K, tn), lambda i, j: (i, 0, 0, j)),
        ],
        out_specs=pl.BlockSpec((1, 21, 3, tn), lambda i, j: (i, 0, 0, j)),
        out_shape=jax.ShapeDtypeStruct((b, 21, 3, n), jnp.float32),
    )(wblk, wbblk, x, fd)


def _edge_call(idx_flat, g):
    b = idx_flat.shape[0]
    n = g.shape[3]
    ntask = b * 21
    mesh = plsc.VectorSubcoreMesh(core_axis_name="c", subcore_axis_name="s")

    @functools.partial(
        pl.kernel,
        out_type=jax.ShapeDtypeStruct((b, 21, 3, n), jnp.float32),
        mesh=mesh,
        scratch_types=[
            pltpu.VMEM((n * K,), jnp.int32),
            pltpu.VMEM((4, 3, n), jnp.float32),
            pltpu.VMEM((3, n), jnp.float32),
        ],
        compiler_params=pltpu.CompilerParams(needs_layout_passes=False),
    )
    def edge(idx_hbm, g_hbm, h_hbm, idx_v, table_v, out_v):
        w = lax.axis_index("s") * NC + lax.axis_index("c")
        t0 = (w * ntask) // NW
        t1 = ((w + 1) * ntask) // NW
        zero = jnp.zeros((L,), jnp.int32)
        one = jnp.full((L,), 1, jnp.int32)
        two = jnp.full((L,), 2, jnp.int32)
        iota = lax.iota(jnp.int32, L)

        def task(t, carry):
            bb = t // 21
            oo = t % 21
            pltpu.sync_copy(idx_hbm.at[bb], idx_v)
            for blk in range(4):
                pltpu.sync_copy(g_hbm.at[bb, blk * 21 + oo], table_v.at[blk])

            def grp(gg, carry2):
                nsl = pl.ds(gg * L, L)
                ebase = (gg * L + iota) * K
                q0 = table_v[2, 0, nsl]
                q1 = table_v[2, 1, nsl]
                q2 = table_v[2, 2, nsl]
                e0 = table_v[3, 0, nsl]
                e1 = table_v[3, 1, nsl]
                e2 = table_v[3, 2, nsl]
                a0 = jnp.zeros((L,), jnp.float32)
                a1 = jnp.zeros((L,), jnp.float32)
                a2 = jnp.zeros((L,), jnp.float32)
                for k in range(K):
                    j = plsc.load_gather(idx_v, [ebase + k])
                    p0 = plsc.load_gather(table_v, [zero, zero, j]) + q0
                    p1 = plsc.load_gather(table_v, [zero, one, j]) + q1
                    p2 = plsc.load_gather(table_v, [zero, two, j]) + q2
                    d0 = plsc.load_gather(table_v, [one, zero, j]) + e0
                    d1 = plsc.load_gather(table_v, [one, one, j]) + e1
                    d2 = plsc.load_gather(table_v, [one, two, j]) + e2
                    dot = p0 * d0 + p1 * d1 + p2 * d2
                    dsq = d0 * d0 + d1 * d1 + d2 * d2
                    coef = NEG * jnp.minimum(dot, 0.0) / (dsq + EPS)
                    a0 = a0 + (p0 - coef * d0)
                    a1 = a1 + (p1 - coef * d1)
                    a2 = a2 + (p2 - coef * d2)
                out_v[0, nsl] = a0 * (1.0 / K)
                out_v[1, nsl] = a1 * (1.0 / K)
                out_v[2, nsl] = a2 * (1.0 / K)
                return carry2

            lax.fori_loop(0, n // L, grp, 0)
            pltpu.sync_copy(out_v, h_hbm.at[bb, oo])
            return carry

        lax.fori_loop(t0, t1, task, 0)

    return edge(idx_flat, g)


def _final_body(x_ref, h1_ref, h2_ref, wf_ref, wd_ref, out_ref):
    xc = [jnp.concatenate(
        [x_ref[0, :, v, :], h1_ref[0, :, v, :], h2_ref[0, :, v, :]], axis=0)
        for v in range(3)]
    for o0, co in ((0, 8), (8, 8), (16, 8), (24, 8), (32, 8), (40, 2)):
        ps, ds = [], []
        dot = jnp.zeros((co, x_ref.shape[3]), jnp.float32)
        dsq = jnp.zeros((co, x_ref.shape[3]), jnp.float32)
        for v in range(3):
            pv = jax.lax.dot_general(
                wf_ref[o0:o0 + co, :], xc[v],
                (((1,), (0,)), ((), ())),
                preferred_element_type=jnp.float32,
                precision=jax.lax.Precision.DEFAULT)
            dv = jax.lax.dot_general(
                wd_ref[o0:o0 + co, :], xc[v],
                (((1,), (0,)), ((), ())),
                preferred_element_type=jnp.float32,
                precision=jax.lax.Precision.DEFAULT)
            ps.append(pv)
            ds.append(dv)
            dot = dot + pv * dv
            dsq = dsq + dv * dv
        coef = NEG * jnp.minimum(dot, 0.0) / (dsq + EPS)
        for v in range(3):
            out_ref[0, o0:o0 + co, v, :] = ps[v] - coef * ds[v]


def _final(x, h1, h2, wlf, wld):
    b, _, _, n = x.shape
    return pl.pallas_call(
        _final_body,
        grid=(b,),
        in_specs=[
            pl.BlockSpec((1, 4, 3, n), lambda i: (i, 0, 0, 0)),
            pl.BlockSpec((1, 21, 3, n), lambda i: (i, 0, 0, 0)),
            pl.BlockSpec((1, 21, 3, n), lambda i: (i, 0, 0, 0)),
            pl.BlockSpec((42, 46), lambda i: (0, 0)),
            pl.BlockSpec((42, 46), lambda i: (0, 0)),
        ],
        out_specs=pl.BlockSpec((1, 42, 3, n), lambda i: (i, 0, 0, 0)),
        out_shape=jax.ShapeDtypeStruct((b, 42, 3, n), jnp.float32),
    )(x, h1, h2, wlf, wld)


def _mix(wf, wd, c):
    return jnp.concatenate(
        [wf[:, :c], wd[:, :c], wf[:, c:] - wf[:, :c], wd[:, c:] - wd[:, :c]],
        axis=0)


def _vnblk(wa, wd):
    m = jnp.concatenate([wa, wd], axis=0)
    c = wa.shape[1]
    cols = [u * c + cc for cc in range(c) for u in range(3)]
    return jnp.kron(jnp.eye(3, dtype=jnp.float32), m)[:, cols]


def kernel(x, W1f, W1d, W2f, W2d, Wlf, Wld):
    b, c, _, n = x.shape
    xf1 = x.reshape(b, 12, n)
    idx1 = _topk(jnp.pad(xf1, ((0, 0), (0, 4), (0, 0))))
    fd1 = _gather_call(idx1.reshape(b, n * K), xf1, nsplit=4)
    wblk = _vnblk(W1f[:, :4], W1d[:, :4])
    wbblk = _vnblk(W1f[:, 4:], W1d[:, 4:])
    h1 = _edge1(x, fd1, wblk, wbblk)
    xf2 = jnp.pad(h1.reshape(b, 63, n), ((0, 0), (0, 1), (0, 0)))
    idx2 = _topk(xf2)
    g2 = _precompute(h1, _mix(W2f, W2d, 21))
    h2 = _edge_call(idx2.reshape(b, n * K), g2)
    return _final(x, h1, h2, Wlf, Wld)


# speedup vs baseline: 8.9195x; 1.0884x over previous
import functools
import jax
import jax.numpy as jnp
from jax import lax
from jax.experimental import pallas as pl
from jax.experimental.pallas import tpu as pltpu
from jax.experimental.pallas import tpu_sc as plsc

K = 20
NEG = 0.8
EPS = 1e-6
NC, NS, L = 2, 16, 16
NW = NC * NS


def _topk_body(xff_ref, xfb_ref, idx_ref):
    xff = xff_ref[0]
    xfb = xfb_ref[0]
    tn = xfb.shape[1]
    n = xff.shape[1]
    inner = jax.lax.dot_general(
        xfb, xff, (((0,), (0,)), ((), ())),
        preferred_element_type=jnp.float32,
        precision=jax.lax.Precision.DEFAULT)
    xx = jnp.sum(xff * xff, axis=0, keepdims=True)
    xxr = jnp.sum(xfb * xfb, axis=0)[:, None]
    work = (-xx - (-2.0 * inner)) - xxr
    iota = lax.broadcasted_iota(jnp.int32, (tn, n), 1)
    cols = []
    m = jnp.max(work, axis=1, keepdims=True)
    for t in range(K):
        cand = jnp.min(jnp.where(work == m, iota, n), axis=1, keepdims=True)
        cols.append(cand)
        if t < K - 1:
            m = jnp.max(jnp.where(work >= m, -jnp.inf, work),
                        axis=1, keepdims=True)
    idx_ref[0] = jnp.concatenate(cols, axis=1)


def _topk(xfp, tn=256):
    b, cp, n = xfp.shape
    grid = (b, n // tn)
    return pl.pallas_call(
        _topk_body,
        grid=grid,
        in_specs=[
            pl.BlockSpec((1, cp, n), lambda i, j: (i, 0, 0)),
            pl.BlockSpec((1, cp, tn), lambda i, j: (i, 0, j)),
        ],
        out_specs=pl.BlockSpec((1, tn, K), lambda i, j: (i, j, 0)),
        out_shape=jax.ShapeDtypeStruct((b, n, K), jnp.int32),
    )(xfp, xfp)


def _pre_body(m_ref, x_ref, g_ref):
    for v in range(3):
        xv = x_ref[0, :, v, :]
        g_ref[0, :, v, :] = jax.lax.dot_general(
            m_ref[...], xv, (((1,), (0,)), ((), ())),
            preferred_element_type=jnp.float32,
            precision=jax.lax.Precision.DEFAULT)


def _precompute(xs, m):
    b, c, _, n = xs.shape
    return pl.pallas_call(
        _pre_body,
        grid=(b,),
        in_specs=[
            pl.BlockSpec((84, c), lambda i: (0, 0)),
            pl.BlockSpec((1, c, 3, n), lambda i: (i, 0, 0, 0)),
        ],
        out_specs=pl.BlockSpec((1, 84, 3, n), lambda i: (i, 0, 0, 0)),
        out_shape=jax.ShapeDtypeStruct((b, 84, 3, n), jnp.float32),
    )(m, xs)


def _gather_call(idx_flat, xr, nsplit):
    b, r, n = xr.shape
    nseg = n // nsplit
    ntask = b * nsplit
    mesh = plsc.VectorSubcoreMesh(core_axis_name="c", subcore_axis_name="s")

    @functools.partial(
        pl.kernel,
        out_type=jax.ShapeDtypeStruct((b, r, K, n), jnp.float32),
        mesh=mesh,
        scratch_types=[
            pltpu.VMEM((nseg * K,), jnp.int32),
            pltpu.VMEM((r, n), jnp.float32),
            pltpu.VMEM((r, nseg), jnp.float32),
        ],
        compiler_params=pltpu.CompilerParams(needs_layout_passes=False),
    )
    def gat(idx_hbm, xr_hbm, fd_hbm, idx_v, table_v, stage_v):
        w = lax.axis_index("s") * NC + lax.axis_index("c")
        t0 = (w * ntask) // NW
        t1 = ((w + 1) * ntask) // NW
        iota = lax.iota(jnp.int32, L)

        def task(t, carry):
            bb = t // nsplit
            n0 = (t % nsplit) * nseg
            pltpu.sync_copy(idx_hbm.at[bb, pl.ds(n0 * K, nseg * K)], idx_v)
            pltpu.sync_copy(xr_hbm.at[bb], table_v)

            def per_k(k, carry2):
                def grp(gg, carry3):
                    jv = plsc.load_gather(idx_v, [(gg * L + iota) * K + k])
                    lsl = pl.ds(gg * L, L)
                    gsl = pl.ds(n0 + gg * L, L)
                    for rr in range(r):
                        gv = plsc.load_gather(
                            table_v, [jnp.full((L,), rr, jnp.int32), jv])
                        stage_v[rr, lsl] = gv - table_v[rr, gsl]
                    return carry3

                lax.fori_loop(0, nseg // L, grp, 0)
                pltpu.sync_copy(stage_v, fd_hbm.at[bb, :, k, pl.ds(n0, nseg)])
                return carry2

            lax.fori_loop(0, K, per_k, 0)
            return carry

        lax.fori_loop(t0, t1, task, 0)

    return gat(idx_flat, xr)


def _edge1_body(wblk_ref, wbblk_ref, x_ref, fd_ref, h_ref):
    tn = x_ref.shape[3]
    r3 = x_ref.shape[1] * 3
    xall = x_ref[0].reshape(r3, tn)
    pbdb = jax.lax.dot_general(
        wbblk_ref[...], xall, (((1,), (0,)), ((), ())),
        preferred_element_type=jnp.float32,
        precision=jax.lax.Precision.DEFAULT)
    acc = [jnp.zeros((21, tn), jnp.float32) for _ in range(3)]
    for k in range(K):
        fd = fd_ref[0, :, k, :]
        pada = jax.lax.dot_general(
            wblk_ref[...], fd, (((1,), (0,)), ((), ())),
            preferred_element_type=jnp.float32,
            precision=jax.lax.Precision.DEFAULT) + pbdb
        ps = [pada[v * 42:v * 42 + 21, :] for v in range(3)]
        ds = [pada[v * 42 + 21:v * 42 + 42, :] for v in range(3)]
        dot = (ps[0] * ds[0] + ps[1] * ds[1]) + ps[2] * ds[2]
        dsq = (ds[0] * ds[0] + ds[1] * ds[1]) + ds[2] * ds[2]
        dotq = dot / (dsq + EPS)
        msk = (dot >= 0.0).astype(jnp.float32)
        for v in range(3):
            inner = msk * ps[v] + (1.0 - msk) * (ps[v] - dotq * ds[v])
            acc[v] = acc[v] + (0.2 * ps[v] + 0.8 * inner)
    for v in range(3):
        h_ref[0, :, v, :] = acc[v] / jnp.float32(K)


def _edge1(x, fd, wblk, wbblk):
    b, _, _, n = x.shape
    tn = 256
    return pl.pallas_call(
        _edge1_body,
        grid=(b, n // tn),
        in_specs=[
            pl.BlockSpec((126, 12), lambda i, j: (0, 0)),
            pl.BlockSpec((126, 12), lambda i, j: (0, 0)),
            pl.BlockSpec((1, 4, 3, tn), lambda i, j: (i, 0, 0, j)),
            pl.BlockSpec((1, 12, K, tn), lambda i, j: (i, 0, 0, j)),
        ],
        out_specs=pl.BlockSpec((1, 21, 3, tn), lambda i, j: (i, 0, 0, j)),
        out_shape=jax.ShapeDtypeStruct((b, 21, 3, n), jnp.float32),
    )(wblk, wbblk, x, fd)


def _edge_call(idx_flat, g):
    b = idx_flat.shape[0]
    n = g.shape[3]
    ntask = b * 21
    mesh = plsc.VectorSubcoreMesh(core_axis_name="c", subcore_axis_name="s")

    @functools.partial(
        pl.kernel,
        out_type=jax.ShapeDtypeStruct((b, 21, 3, n), jnp.float32),
        mesh=mesh,
        scratch_types=[
            pltpu.VMEM((n * K,), jnp.int32),
            pltpu.VMEM((4, 3, n), jnp.float32),
            pltpu.VMEM((3, n), jnp.float32),
        ],
        compiler_params=pltpu.CompilerParams(needs_layout_passes=False),
    )
    def edge(idx_hbm, g_hbm, h_hbm, idx_v, table_v, out_v):
        w = lax.axis_index("s") * NC + lax.axis_index("c")
        t0 = (w * ntask) // NW
        t1 = ((w + 1) * ntask) // NW
        zero = jnp.zeros((L,), jnp.int32)
        one = jnp.full((L,), 1, jnp.int32)
        two = jnp.full((L,), 2, jnp.int32)
        iota = lax.iota(jnp.int32, L)

        def task(t, carry):
            bb = t // 21
            oo = t % 21
            pltpu.sync_copy(idx_hbm.at[bb], idx_v)
            for blk in range(4):
                pltpu.sync_copy(g_hbm.at[bb, blk * 21 + oo], table_v.at[blk])

            def grp(gg, carry2):
                nsl = pl.ds(gg * L, L)
                ebase = (gg * L + iota) * K
                q0 = table_v[2, 0, nsl]
                q1 = table_v[2, 1, nsl]
                q2 = table_v[2, 2, nsl]
                e0 = table_v[3, 0, nsl]
                e1 = table_v[3, 1, nsl]
                e2 = table_v[3, 2, nsl]
                a0 = jnp.zeros((L,), jnp.float32)
                a1 = jnp.zeros((L,), jnp.float32)
                a2 = jnp.zeros((L,), jnp.float32)
                for k in range(K):
                    j = plsc.load_gather(idx_v, [ebase + k])
                    p0 = plsc.load_gather(table_v, [zero, zero, j]) + q0
                    p1 = plsc.load_gather(table_v, [zero, one, j]) + q1
                    p2 = plsc.load_gather(table_v, [zero, two, j]) + q2
                    d0 = plsc.load_gather(table_v, [one, zero, j]) + e0
                    d1 = plsc.load_gather(table_v, [one, one, j]) + e1
                    d2 = plsc.load_gather(table_v, [one, two, j]) + e2
                    dot = p0 * d0 + p1 * d1 + p2 * d2
                    dsq = d0 * d0 + d1 * d1 + d2 * d2
                    coef = NEG * jnp.minimum(dot, 0.0) / (dsq + EPS)
                    a0 = a0 + (p0 - coef * d0)
                    a1 = a1 + (p1 - coef * d1)
                    a2 = a2 + (p2 - coef * d2)
                out_v[0, nsl] = a0 * (1.0 / K)
                out_v[1, nsl] = a1 * (1.0 / K)
                out_v[2, nsl] = a2 * (1.0 / K)
                return carry2

            lax.fori_loop(0, n // L, grp, 0)
            pltpu.sync_copy(out_v, h_hbm.at[bb, oo])
            return carry

        lax.fori_loop(t0, t1, task, 0)

    return edge(idx_flat, g)


def _final_body(x_ref, h1_ref, h2_ref, wf_ref, wd_ref, out_ref):
    xc = [jnp.concatenate(
        [x_ref[0, :, v, :], h1_ref[0, :, v, :], h2_ref[0, :, v, :]], axis=0)
        for v in range(3)]
    for o0, co in ((0, 8), (8, 8), (16, 8), (24, 8), (32, 8), (40, 2)):
        ps, ds = [], []
        dot = jnp.zeros((co, x_ref.shape[3]), jnp.float32)
        dsq = jnp.zeros((co, x_ref.shape[3]), jnp.float32)
        for v in range(3):
            pv = jax.lax.dot_general(
                wf_ref[o0:o0 + co, :], xc[v],
                (((1,), (0,)), ((), ())),
                preferred_element_type=jnp.float32,
                precision=jax.lax.Precision.DEFAULT)
            dv = jax.lax.dot_general(
                wd_ref[o0:o0 + co, :], xc[v],
                (((1,), (0,)), ((), ())),
                preferred_element_type=jnp.float32,
                precision=jax.lax.Precision.DEFAULT)
            ps.append(pv)
            ds.append(dv)
            dot = dot + pv * dv
            dsq = dsq + dv * dv
        coef = NEG * jnp.minimum(dot, 0.0) / (dsq + EPS)
        for v in range(3):
            out_ref[0, o0:o0 + co, v, :] = ps[v] - coef * ds[v]


def _final(x, h1, h2, wlf, wld):
    b, _, _, n = x.shape
    return pl.pallas_call(
        _final_body,
        grid=(b,),
        in_specs=[
            pl.BlockSpec((1, 4, 3, n), lambda i: (i, 0, 0, 0)),
            pl.BlockSpec((1, 21, 3, n), lambda i: (i, 0, 0, 0)),
            pl.BlockSpec((1, 21, 3, n), lambda i: (i, 0, 0, 0)),
            pl.BlockSpec((42, 46), lambda i: (0, 0)),
            pl.BlockSpec((42, 46), lambda i: (0, 0)),
        ],
        out_specs=pl.BlockSpec((1, 42, 3, n), lambda i: (i, 0, 0, 0)),
        out_shape=jax.ShapeDtypeStruct((b, 42, 3, n), jnp.float32),
    )(x, h1, h2, wlf, wld)


def _mix(wf, wd, c):
    return jnp.concatenate(
        [wf[:, :c], wd[:, :c], wf[:, c:] - wf[:, :c], wd[:, c:] - wd[:, :c]],
        axis=0)


def _vnblk(wa, wd):
    m = jnp.concatenate([wa, wd], axis=0)
    c = wa.shape[1]
    cols = [u * c + cc for cc in range(c) for u in range(3)]
    return jnp.kron(jnp.eye(3, dtype=jnp.float32), m)[:, cols]


def kernel(x, W1f, W1d, W2f, W2d, Wlf, Wld):
    b, c, _, n = x.shape
    xf1 = x.reshape(b, 12, n)
    idx1 = _topk(jnp.pad(xf1, ((0, 0), (0, 4), (0, 0))))
    fd1 = _gather_call(idx1.reshape(b, n * K), xf1, nsplit=4)
    wblk = _vnblk(W1f[:, :4], W1d[:, :4])
    wbblk = _vnblk(W1f[:, 4:], W1d[:, 4:])
    h1 = _edge1(x, fd1, wblk, wbblk)
    xf2 = jnp.pad(h1.reshape(b, 63, n), ((0, 0), (0, 1), (0, 0)))
    idx2 = _topk(xf2)
    g2 = _precompute(h1, _mix(W2f, W2d, 21))
    h2 = _edge_call(idx2.reshape(b, n * K), g2)
    return _final(x, h1, h2, Wlf, Wld)
